# baseline (speedup 1.0000x reference)
import functools

import jax
import jax.numpy as jnp
from jax import lax
from jax.experimental import pallas as pl
from jax.experimental.pallas import tpu as pltpu
from jax.experimental.pallas import tpu_sc as plsc

_LANE = 128
_BM = 512
_BM2 = 1024
_BC = 1024
_NEG = -1.0e30
_F32 = jnp.float32


def _dot(a, b):
    return jnp.dot(a, b, preferred_element_type=jnp.float32)


def _kdot(a, b):
    kdim = a.shape[1]
    acc = None
    for k in range(0, kdim, 256):
        part = jnp.dot(a[:, k:k + 256], b[k:k + 256, :],
                       preferred_element_type=jnp.float32)
        acc = part if acc is None else acc + part
    return acc


def _colsum(x):
    accs = [None] * 16
    nt = x.shape[0] // 8
    for k in range(nt):
        u = k % 16
        blk = x[8 * k:8 * (k + 1), :]
        accs[u] = blk if accs[u] is None else accs[u] + blk
    acc = accs[0]
    for u in range(1, 16):
        acc = acc + accs[u]
    s = acc[0:4] + acc[4:8]
    s = s[0:2] + s[2:4]
    return s[0] + s[1]


def _bn_cols(h, g, b):
    n = h.shape[0]
    mu = _colsum(h) / float(n)
    var = _colsum((h - mu[None, :]) ** 2) / float(n)
    return g[None, :] * (h - mu[None, :]) / jnp.sqrt(var[None, :] + 1e-5) + b[None, :]


def _k1_body(a1_ref, a2_ref, a3_ref, x_ref, w_ref, b_ref, adj_ref, h1_ref, x1_scr):
    j = pl.program_id(1)

    @pl.when(j == 0)
    def _():
        x1_scr[...] = _dot(x_ref[0], w_ref[...])

    adj = jnp.maximum(a1_ref[0] + a2_ref[0] + a3_ref[0], 0.0)
    adj_ref[0] = adj
    h1_ref[0] = jnp.maximum(_kdot(adj, x1_scr[...]) + b_ref[...][None, :], 0.0)


def _k2_body(adj_ref, h1_ref, w_ref, b_ref, ls_ref, x2_scr):
    j = pl.program_id(1)

    @pl.when(j == 0)
    def _():
        x2_scr[...] = _dot(h1_ref[0], w_ref[...])

    h = _kdot(adj_ref[0], x2_scr[...]) + b_ref[...][None, :]
    m = jnp.max(h, axis=1, keepdims=True)
    e = h - m
    ls_ref[0] = e - jnp.log(jnp.sum(jnp.exp(e), axis=1, keepdims=True))


def _k3_body(ls_ref, g_ref, b_ref, w_ref, xs_ref, x3_ref):
    ne = _bn_cols(ls_ref[0], g_ref[...], b_ref[...])
    xs_ref[0] = ne
    x3_ref[0] = _dot(ne, w_ref[...])


def _k4_body(adj_ref, x3_ref, gb_ref, mw_ref, mb_ref, mc_ref):
    mc1 = jnp.maximum(_kdot(adj_ref[0], x3_ref[0]) + gb_ref[...][None, :], 0.0)
    mc_ref[0] = jnp.maximum(_dot(mc1, mw_ref[...]) + mb_ref[...][None, :], 0.0)


def _k5_body(mc_ref, g_ref, b_ref, cw_ref, cb_ref, cs_ref, ids_ref):
    n = mc_ref.shape[1]
    ncl = 25
    mc = mc_ref[0]
    mcbn = _bn_cols(mc, g_ref[...], b_ref[...])
    lane = lax.broadcasted_iota(jnp.int32, (n, _LANE), 1)
    masked = jnp.where(lane < ncl, mcbn, _NEG)
    mv = jnp.max(masked, axis=1, keepdims=True)
    ids = jnp.min(jnp.where(masked == mv, lane, 1 << 30), axis=1,
                  keepdims=True)
    ss = jax.nn.sigmoid(_dot(mcbn, cw_ref[...]) + cb_ref[0, 0])
    onehot = (ids == lane).astype(_F32)
    counts = jnp.sum(onehot, axis=0, keepdims=True)
    sums = lax.dot_general(onehot, ss, (((0,), (0,)), ((), ())),
                           preferred_element_type=jnp.float32)
    score = sums[:, 0][None, :] / jnp.maximum(counts, 1.0)
    cs_ref[0] = score
    ids_ref[0] = ids.reshape(1, n)


def _make_sc_select(b, n, ncl):
    nw = 32
    cpg = nw // b
    chunk = n // cpg
    nvr = chunk // 16

    def body(sc_hbm, ids_hbm, cid_out, tm_out, sc_v, id_v, tm_v, cid_v):
        c = lax.axis_index("c")
        s = lax.axis_index("s")
        wid = s * 2 + c
        g = wid // cpg
        ch = wid % cpg
        pltpu.sync_copy(sc_hbm.at[pl.ds(g * _LANE, _LANE)], sc_v)
        vr0 = sc_v[pl.ds(0, 16)]
        vr1 = sc_v[pl.ds(16, 16)]
        svals = [vr0[i] for i in range(16)] + [vr1[i] for i in range(ncl - 16)]

        def top1(exclude):
            bi = jnp.int32(0)
            bv = jnp.float32(-3e38)
            for i in range(ncl):
                v = svals[i]
                if exclude is not None:
                    v = jnp.where(exclude == i, jnp.float32(-3e38), v)
                take = v > bv
                bi = jnp.where(take, jnp.int32(i), bi)
                bv = jnp.where(take, v, bv)
            return bi

        i1 = top1(None)
        i2 = top1(i1)
        for j in range(_LANE // 16):
            idx = lax.iota(jnp.int32, 16) + 16 * j
            cid_v[pl.ds(16 * j, 16)] = jnp.where(
                idx == 0, i1, jnp.where(idx == 1, i2, 0))

        @pl.when(ch == 0)
        def _():
            pltpu.sync_copy(cid_v, cid_out.at[pl.ds(g * _LANE, _LANE)])

        pltpu.sync_copy(ids_hbm.at[pl.ds(g * n + ch * chunk, chunk)], id_v)
        for j in range(nvr):
            v = id_v[pl.ds(16 * j, 16)]
            tm_v[pl.ds(16 * j, 16)] = jnp.where(jnp.logical_or(v == i1, v == i2), 1.0, 0.0)
        pltpu.sync_copy(tm_v, tm_out.at[pl.ds(g * n + ch * chunk, chunk)])

    mesh = plsc.VectorSubcoreMesh(core_axis_name="c", subcore_axis_name="s")
    return pl.kernel(
        body, mesh=mesh,
        out_type=[jax.ShapeDtypeStruct((b * _LANE,), jnp.int32),
                  jax.ShapeDtypeStruct((b * n,), jnp.float32)],
        scratch_types=[pltpu.VMEM((_LANE,), jnp.float32),
                       pltpu.VMEM((chunk,), jnp.int32),
                       pltpu.VMEM((chunk,), jnp.float32),
                       pltpu.VMEM((_LANE,), jnp.int32)])


def _k6_body(xsb_ref, xsf_ref, st_ref, out_ref):
    sblk = jnp.maximum(
        lax.dot_general(xsb_ref[0], xsf_ref[0], (((1,), (1,)), ((), ())),
                        preferred_element_type=jnp.float32), 0.0)
    out_ref[0, 0] = jnp.sum(sblk * st_ref[0], axis=1, keepdims=True).reshape(1, -1)


def _bn_rows(p, g, b):
    mu = jnp.mean(p, axis=0, keepdims=True)
    var = jnp.mean((p - mu) ** 2, axis=0, keepdims=True)
    return g[None, :] * (p - mu) / jnp.sqrt(var + 1e-5) + b[None, :]


def _k7a_body(p_ref, g1_ref, b1_ref, w_ref, wb_ref, g2_ref, b2_ref, out_ref):
    z = _bn_rows(p_ref[...], g1_ref[...], b1_ref[...])
    y = jnp.maximum(_dot(z, w_ref[...]) + wb_ref[...][None, :], 0.0)
    out_ref[...] = _bn_rows(y, g2_ref[...], b2_ref[...])


def _k7b_body(y_ref, w_ref, b_ref, out_ref):
    out_ref[...] = jax.nn.sigmoid(_dot(y_ref[...], w_ref[...]) + b_ref[...][None, :])


def kernel(x, a1, a2, a3, state, gc1_w, gc1_b, gc2_w, gc2_b, norm_g, norm_b,
           cl_gc_w, cl_gc_b, cl_mlp_w, cl_mlp_b, cln_g, cln_b, cent_w, cent_b,
           norm1_g, norm1_b, mlp1_w, mlp1_b, norm2_g, norm2_b, mlp2_w, mlp2_b):
    b, n, f = x.shape
    nhid = gc1_w.shape[1]
    ncls = gc2_w.shape[1]
    hid = cl_gc_w.shape[1]
    ncl = cl_mlp_w.shape[1]
    nb = n // _BM
    f32 = jnp.float32

    adj, h1 = pl.pallas_call(
        _k1_body,
        grid=(b, nb),
        in_specs=[
            pl.BlockSpec((1, _BM, n), lambda t, j: (t, j, 0)),
            pl.BlockSpec((1, _BM, n), lambda t, j: (t, j, 0)),
            pl.BlockSpec((1, _BM, n), lambda t, j: (t, j, 0)),
            pl.BlockSpec((1, n, f), lambda t, j: (t, 0, 0)),
            pl.BlockSpec((f, nhid), lambda t, j: (0, 0)),
            pl.BlockSpec((nhid,), lambda t, j: (0,)),
        ],
        out_specs=[
            pl.BlockSpec((1, _BM, n), lambda t, j: (t, j, 0)),
            pl.BlockSpec((1, _BM, nhid), lambda t, j: (t, j, 0)),
        ],
        out_shape=[
            jax.ShapeDtypeStruct((b, n, n), f32),
            jax.ShapeDtypeStruct((b, n, nhid), f32),
        ],
        scratch_shapes=[pltpu.VMEM((n, nhid), f32)],
    )(a1, a2, a3, x, gc1_w, gc1_b)

    nb2 = n // _BM2
    ls = pl.pallas_call(
        _k2_body,
        grid=(b, nb2),
        in_specs=[
            pl.BlockSpec((1, _BM2, n), lambda t, j: (t, j, 0)),
            pl.BlockSpec((1, n, nhid), lambda t, j: (t, 0, 0)),
            pl.BlockSpec((nhid, ncls), lambda t, j: (0, 0)),
            pl.BlockSpec((ncls,), lambda t, j: (0,)),
        ],
        out_specs=pl.BlockSpec((1, _BM2, ncls), lambda t, j: (t, j, 0)),
        out_shape=jax.ShapeDtypeStruct((b, n, ncls), f32),
        scratch_shapes=[pltpu.VMEM((n, ncls), f32)],
    )(adj, h1, gc2_w, gc2_b)

    xs, x3 = pl.pallas_call(
        _k3_body,
        grid=(b,),
        in_specs=[
            pl.BlockSpec((1, n, ncls), lambda t: (t, 0, 0)),
            pl.BlockSpec((ncls,), lambda t: (0,)),
            pl.BlockSpec((ncls,), lambda t: (0,)),
            pl.BlockSpec((ncls, hid), lambda t: (0, 0)),
        ],
        out_specs=[
            pl.BlockSpec((1, n, ncls), lambda t: (t, 0, 0)),
            pl.BlockSpec((1, n, hid), lambda t: (t, 0, 0)),
        ],
        out_shape=[
            jax.ShapeDtypeStruct((b, n, ncls), f32),
            jax.ShapeDtypeStruct((b, n, hid), f32),
        ],
    )(ls, norm_g, norm_b, cl_gc_w)

    mlp_w_p = jnp.zeros((hid, _LANE), f32).at[:, :ncl].set(cl_mlp_w)
    mlp_b_p = jnp.zeros((_LANE,), f32).at[:ncl].set(cl_mlp_b)
    mc = pl.pallas_call(
        _k4_body,
        grid=(b, nb2),
        in_specs=[
            pl.BlockSpec((1, _BM2, n), lambda t, j: (t, j, 0)),
            pl.BlockSpec((1, n, hid), lambda t, j: (t, 0, 0)),
            pl.BlockSpec((hid,), lambda t, j: (0,)),
            pl.BlockSpec((hid, _LANE), lambda t, j: (0, 0)),
            pl.BlockSpec((_LANE,), lambda t, j: (0,)),
        ],
        out_specs=pl.BlockSpec((1, _BM2, _LANE), lambda t, j: (t, j, 0)),
        out_shape=jax.ShapeDtypeStruct((b, n, _LANE), f32),
    )(adj, x3, cl_gc_b, mlp_w_p, mlp_b_p)

    cln_g_p = jnp.ones((_LANE,), f32).at[:ncl].set(cln_g)
    cln_b_p = jnp.zeros((_LANE,), f32).at[:ncl].set(cln_b)
    cent_w_p = jnp.zeros((_LANE, 1), f32).at[:ncl, 0].set(cent_w[:, 0])
    cent_b_p = jnp.full((1, _LANE), cent_b[0], f32)
    cs_p, ids_p = pl.pallas_call(
        _k5_body,
        grid=(b,),
        in_specs=[
            pl.BlockSpec((1, n, _LANE), lambda t: (t, 0, 0)),
            pl.BlockSpec((_LANE,), lambda t: (0,)),
            pl.BlockSpec((_LANE,), lambda t: (0,)),
            pl.BlockSpec((_LANE, 1), lambda t: (0, 0)),
            pl.BlockSpec((1, _LANE), lambda t: (0, 0)),
        ],
        out_specs=[
            pl.BlockSpec((1, 1, _LANE), lambda t: (t, 0, 0)),
            pl.BlockSpec((1, 1, n), lambda t: (t, 0, 0)),
        ],
        out_shape=[
            jax.ShapeDtypeStruct((b, 1, _LANE), f32),
            jax.ShapeDtypeStruct((b, 1, n), jnp.int32),
        ],
    )(mc, cln_g_p, cln_b_p, cent_w_p, cent_b_p)

    cid_flat, tm_flat = _make_sc_select(b, n, ncl)(
        cs_p.reshape(b * _LANE), ids_p.reshape(b * n))

    state_t = state.reshape(b, 1, n)
    pooled_blk = pl.pallas_call(
        _k6_body,
        grid=(b, nb2),
        in_specs=[
            pl.BlockSpec((1, _BM2, ncls), lambda t, j: (t, j, 0)),
            pl.BlockSpec((1, n, ncls), lambda t, j: (t, 0, 0)),
            pl.BlockSpec((1, 1, n), lambda t, j: (t, 0, 0)),
        ],
        out_specs=pl.BlockSpec((1, 1, 1, _BM2), lambda t, j: (t, j, 0, 0)),
        out_shape=jax.ShapeDtypeStruct((b, nb2, 1, _BM2), f32),
    )(xs, xs, state_t)
    pooled = pooled_blk.reshape(b, n)

    d1 = mlp1_w.shape[1]
    y1 = pl.pallas_call(
        _k7a_body,
        grid=(d1 // _BC,),
        in_specs=[
            pl.BlockSpec((b, n), lambda c: (0, 0)),
            pl.BlockSpec((n,), lambda c: (0,)),
            pl.BlockSpec((n,), lambda c: (0,)),
            pl.BlockSpec((n, _BC), lambda c: (0, c)),
            pl.BlockSpec((_BC,), lambda c: (c,)),
            pl.BlockSpec((_BC,), lambda c: (c,)),
            pl.BlockSpec((_BC,), lambda c: (c,)),
        ],
        out_specs=pl.BlockSpec((b, _BC), lambda c: (0, c)),
        out_shape=jax.ShapeDtypeStruct((b, d1), f32),
    )(pooled, norm1_g, norm1_b, mlp1_w, mlp1_b, norm2_g, norm2_b)

    d2 = mlp2_w.shape[1]
    out = pl.pallas_call(
        _k7b_body,
        grid=(d2 // _BC,),
        in_specs=[
            pl.BlockSpec((b, d1), lambda c: (0, 0)),
            pl.BlockSpec((d1, _BC), lambda c: (0, c)),
            pl.BlockSpec((_BC,), lambda c: (c,)),
        ],
        out_specs=pl.BlockSpec((b, _BC), lambda c: (0, c)),
        out_shape=jax.ShapeDtypeStruct((b, d2), f32),
    )(y1, mlp2_w, mlp2_b)

    c_s = cs_p[:, 0, :ncl, None]
    t_mask = tm_flat.reshape(b, n, 1)
    c_id = cid_flat.reshape(b, _LANE)[:, :2]
    output = out.reshape(b, n, 2)
    return c_s, t_mask, c_id, output, adj


# speedup vs baseline: 1.0051x; 1.0051x over previous
import functools

import jax
import jax.numpy as jnp
from jax import lax
from jax.experimental import pallas as pl
from jax.experimental.pallas import tpu as pltpu
from jax.experimental.pallas import tpu_sc as plsc

_LANE = 128
_BM = 512
_BM2 = 1024
_BC = 512
_NEG = -1.0e30
_F32 = jnp.float32


def _dot(a, b):
    return jnp.dot(a, b, preferred_element_type=jnp.float32)


def _kdot(a, b):
    kdim = a.shape[1]
    acc = None
    for k in range(0, kdim, 256):
        part = jnp.dot(a[:, k:k + 256], b[k:k + 256, :],
                       preferred_element_type=jnp.float32)
        acc = part if acc is None else acc + part
    return acc


def _colsum(x):
    accs = [None] * 16
    nt = x.shape[0] // 8
    for k in range(nt):
        u = k % 16
        blk = x[8 * k:8 * (k + 1), :]
        accs[u] = blk if accs[u] is None else accs[u] + blk
    acc = accs[0]
    for u in range(1, 16):
        acc = acc + accs[u]
    s = acc[0:4] + acc[4:8]
    s = s[0:2] + s[2:4]
    return s[0] + s[1]


def _bn_cols(h, g, b):
    n = h.shape[0]
    mu = _colsum(h) / float(n)
    var = _colsum((h - mu[None, :]) ** 2) / float(n)
    return g[None, :] * (h - mu[None, :]) / jnp.sqrt(var[None, :] + 1e-5) + b[None, :]


def _k1_body(a1_ref, a2_ref, a3_ref, x_ref, w_ref, b_ref, adj_ref, h1_ref, x1_scr):
    j = pl.program_id(1)

    @pl.when(j == 0)
    def _():
        x1_scr[...] = _dot(x_ref[0], w_ref[...])

    adj = jnp.maximum(a1_ref[0] + a2_ref[0] + a3_ref[0], 0.0)
    adj_ref[0] = adj
    h1_ref[0] = jnp.maximum(_kdot(adj, x1_scr[...]) + b_ref[...][None, :], 0.0)


def _k2_body(adj_ref, h1_ref, w_ref, b_ref, ls_ref, x2_scr):
    j = pl.program_id(1)

    @pl.when(j == 0)
    def _():
        x2_scr[...] = _dot(h1_ref[0], w_ref[...])

    h = _kdot(adj_ref[0], x2_scr[...]) + b_ref[...][None, :]
    m = jnp.max(h, axis=1, keepdims=True)
    e = h - m
    ls_ref[0] = e - jnp.log(jnp.sum(jnp.exp(e), axis=1, keepdims=True))


def _k3_body(ls_ref, g_ref, b_ref, w_ref, xs_ref, x3_ref):
    ne = _bn_cols(ls_ref[0], g_ref[...], b_ref[...])
    xs_ref[0] = ne
    x3_ref[0] = _dot(ne, w_ref[...])


def _k4_body(adj_ref, x3_ref, gb_ref, mw_ref, mb_ref, mc_ref):
    mc1 = jnp.maximum(_kdot(adj_ref[0], x3_ref[0]) + gb_ref[...][None, :], 0.0)
    mc_ref[0] = jnp.maximum(_dot(mc1, mw_ref[...]) + mb_ref[...][None, :], 0.0)


def _k5_body(mc_ref, g_ref, b_ref, cw_ref, cb_ref, cs_ref, ids_ref):
    n = mc_ref.shape[1]
    ncl = 25
    mc = mc_ref[0]
    mcbn = _bn_cols(mc, g_ref[...], b_ref[...])
    lane = lax.broadcasted_iota(jnp.int32, (n, _LANE), 1)
    masked = jnp.where(lane < ncl, mcbn, _NEG)
    mv = jnp.max(masked, axis=1, keepdims=True)
    ids = jnp.min(jnp.where(masked == mv, lane, 1 << 30), axis=1,
                  keepdims=True)
    ss = jax.nn.sigmoid(_dot(mcbn, cw_ref[...]) + cb_ref[0, 0])
    onehot = (ids == lane).astype(_F32)
    counts = jnp.sum(onehot, axis=0, keepdims=True)
    sums = lax.dot_general(onehot, ss, (((0,), (0,)), ((), ())),
                           preferred_element_type=jnp.float32)
    score = sums[:, 0][None, :] / jnp.maximum(counts, 1.0)
    cs_ref[0] = score
    ids_ref[0] = ids.reshape(1, n)


def _make_sc_select(b, n, ncl):
    nw = 32
    cpg = nw // b
    chunk = n // cpg
    nvr = chunk // 16

    def body(sc_hbm, ids_hbm, cid_out, tm_out, sc_v, id_v, tm_v, cid_v):
        c = lax.axis_index("c")
        s = lax.axis_index("s")
        wid = s * 2 + c
        g = wid // cpg
        ch = wid % cpg
        pltpu.sync_copy(sc_hbm.at[pl.ds(g * _LANE, _LANE)], sc_v)
        vr0 = sc_v[pl.ds(0, 16)]
        vr1 = sc_v[pl.ds(16, 16)]
        svals = [vr0[i] for i in range(16)] + [vr1[i] for i in range(ncl - 16)]

        def top1(exclude):
            bi = jnp.int32(0)
            bv = jnp.float32(-3e38)
            for i in range(ncl):
                v = svals[i]
                if exclude is not None:
                    v = jnp.where(exclude == i, jnp.float32(-3e38), v)
                take = v > bv
                bi = jnp.where(take, jnp.int32(i), bi)
                bv = jnp.where(take, v, bv)
            return bi

        i1 = top1(None)
        i2 = top1(i1)
        for j in range(_LANE // 16):
            idx = lax.iota(jnp.int32, 16) + 16 * j
            cid_v[pl.ds(16 * j, 16)] = jnp.where(
                idx == 0, i1, jnp.where(idx == 1, i2, 0))

        @pl.when(ch == 0)
        def _():
            pltpu.sync_copy(cid_v, cid_out.at[pl.ds(g * _LANE, _LANE)])

        pltpu.sync_copy(ids_hbm.at[pl.ds(g * n + ch * chunk, chunk)], id_v)
        for j in range(nvr):
            v = id_v[pl.ds(16 * j, 16)]
            tm_v[pl.ds(16 * j, 16)] = jnp.where(jnp.logical_or(v == i1, v == i2), 1.0, 0.0)
        pltpu.sync_copy(tm_v, tm_out.at[pl.ds(g * n + ch * chunk, chunk)])

    mesh = plsc.VectorSubcoreMesh(core_axis_name="c", subcore_axis_name="s")
    return pl.kernel(
        body, mesh=mesh,
        out_type=[jax.ShapeDtypeStruct((b * _LANE,), jnp.int32),
                  jax.ShapeDtypeStruct((b * n,), jnp.float32)],
        scratch_types=[pltpu.VMEM((_LANE,), jnp.float32),
                       pltpu.VMEM((chunk,), jnp.int32),
                       pltpu.VMEM((chunk,), jnp.float32),
                       pltpu.VMEM((_LANE,), jnp.int32)])


def _k6_body(xsb_ref, xsf_ref, st_ref, out_ref):
    sblk = jnp.maximum(
        lax.dot_general(xsb_ref[0], xsf_ref[0], (((1,), (1,)), ((), ())),
                        preferred_element_type=jnp.float32), 0.0)
    out_ref[0, 0] = jnp.sum(sblk * st_ref[0], axis=1, keepdims=True).reshape(1, -1)


def _bn_rows(p, g, b):
    mu = jnp.mean(p, axis=0, keepdims=True)
    var = jnp.mean((p - mu) ** 2, axis=0, keepdims=True)
    return g[None, :] * (p - mu) / jnp.sqrt(var + 1e-5) + b[None, :]


def _k7a_body(p_ref, g1_ref, b1_ref, w_ref, wb_ref, g2_ref, b2_ref, out_ref):
    z = _bn_rows(p_ref[...], g1_ref[...], b1_ref[...])
    y = jnp.maximum(_dot(z, w_ref[...]) + wb_ref[...][None, :], 0.0)
    out_ref[...] = _bn_rows(y, g2_ref[...], b2_ref[...])


def _k7b_body(y_ref, w_ref, b_ref, out_ref):
    out_ref[...] = jax.nn.sigmoid(_dot(y_ref[...], w_ref[...]) + b_ref[...][None, :])


def kernel(x, a1, a2, a3, state, gc1_w, gc1_b, gc2_w, gc2_b, norm_g, norm_b,
           cl_gc_w, cl_gc_b, cl_mlp_w, cl_mlp_b, cln_g, cln_b, cent_w, cent_b,
           norm1_g, norm1_b, mlp1_w, mlp1_b, norm2_g, norm2_b, mlp2_w, mlp2_b):
    b, n, f = x.shape
    nhid = gc1_w.shape[1]
    ncls = gc2_w.shape[1]
    hid = cl_gc_w.shape[1]
    ncl = cl_mlp_w.shape[1]
    nb = n // _BM
    f32 = jnp.float32

    adj, h1 = pl.pallas_call(
        _k1_body,
        grid=(b, nb),
        in_specs=[
            pl.BlockSpec((1, _BM, n), lambda t, j: (t, j, 0)),
            pl.BlockSpec((1, _BM, n), lambda t, j: (t, j, 0)),
            pl.BlockSpec((1, _BM, n), lambda t, j: (t, j, 0)),
            pl.BlockSpec((1, n, f), lambda t, j: (t, 0, 0)),
            pl.BlockSpec((f, nhid), lambda t, j: (0, 0)),
            pl.BlockSpec((nhid,), lambda t, j: (0,)),
        ],
        out_specs=[
            pl.BlockSpec((1, _BM, n), lambda t, j: (t, j, 0)),
            pl.BlockSpec((1, _BM, nhid), lambda t, j: (t, j, 0)),
        ],
        out_shape=[
            jax.ShapeDtypeStruct((b, n, n), f32),
            jax.ShapeDtypeStruct((b, n, nhid), f32),
        ],
        scratch_shapes=[pltpu.VMEM((n, nhid), f32)],
    )(a1, a2, a3, x, gc1_w, gc1_b)

    nb2 = n // _BM2
    ls = pl.pallas_call(
        _k2_body,
        grid=(b, nb2),
        in_specs=[
            pl.BlockSpec((1, _BM2, n), lambda t, j: (t, j, 0)),
            pl.BlockSpec((1, n, nhid), lambda t, j: (t, 0, 0)),
            pl.BlockSpec((nhid, ncls), lambda t, j: (0, 0)),
            pl.BlockSpec((ncls,), lambda t, j: (0,)),
        ],
        out_specs=pl.BlockSpec((1, _BM2, ncls), lambda t, j: (t, j, 0)),
        out_shape=jax.ShapeDtypeStruct((b, n, ncls), f32),
        scratch_shapes=[pltpu.VMEM((n, ncls), f32)],
    )(adj, h1, gc2_w, gc2_b)

    xs, x3 = pl.pallas_call(
        _k3_body,
        grid=(b,),
        in_specs=[
            pl.BlockSpec((1, n, ncls), lambda t: (t, 0, 0)),
            pl.BlockSpec((ncls,), lambda t: (0,)),
            pl.BlockSpec((ncls,), lambda t: (0,)),
            pl.BlockSpec((ncls, hid), lambda t: (0, 0)),
        ],
        out_specs=[
            pl.BlockSpec((1, n, ncls), lambda t: (t, 0, 0)),
            pl.BlockSpec((1, n, hid), lambda t: (t, 0, 0)),
        ],
        out_shape=[
            jax.ShapeDtypeStruct((b, n, ncls), f32),
            jax.ShapeDtypeStruct((b, n, hid), f32),
        ],
    )(ls, norm_g, norm_b, cl_gc_w)

    mlp_w_p = jnp.zeros((hid, _LANE), f32).at[:, :ncl].set(cl_mlp_w)
    mlp_b_p = jnp.zeros((_LANE,), f32).at[:ncl].set(cl_mlp_b)
    mc = pl.pallas_call(
        _k4_body,
        grid=(b, nb2),
        in_specs=[
            pl.BlockSpec((1, _BM2, n), lambda t, j: (t, j, 0)),
            pl.BlockSpec((1, n, hid), lambda t, j: (t, 0, 0)),
            pl.BlockSpec((hid,), lambda t, j: (0,)),
            pl.BlockSpec((hid, _LANE), lambda t, j: (0, 0)),
            pl.BlockSpec((_LANE,), lambda t, j: (0,)),
        ],
        out_specs=pl.BlockSpec((1, _BM2, _LANE), lambda t, j: (t, j, 0)),
        out_shape=jax.ShapeDtypeStruct((b, n, _LANE), f32),
    )(adj, x3, cl_gc_b, mlp_w_p, mlp_b_p)

    cln_g_p = jnp.ones((_LANE,), f32).at[:ncl].set(cln_g)
    cln_b_p = jnp.zeros((_LANE,), f32).at[:ncl].set(cln_b)
    cent_w_p = jnp.zeros((_LANE, 1), f32).at[:ncl, 0].set(cent_w[:, 0])
    cent_b_p = jnp.full((1, _LANE), cent_b[0], f32)
    cs_p, ids_p = pl.pallas_call(
        _k5_body,
        grid=(b,),
        in_specs=[
            pl.BlockSpec((1, n, _LANE), lambda t: (t, 0, 0)),
            pl.BlockSpec((_LANE,), lambda t: (0,)),
            pl.BlockSpec((_LANE,), lambda t: (0,)),
            pl.BlockSpec((_LANE, 1), lambda t: (0, 0)),
            pl.BlockSpec((1, _LANE), lambda t: (0, 0)),
        ],
        out_specs=[
            pl.BlockSpec((1, 1, _LANE), lambda t: (t, 0, 0)),
            pl.BlockSpec((1, 1, n), lambda t: (t, 0, 0)),
        ],
        out_shape=[
            jax.ShapeDtypeStruct((b, 1, _LANE), f32),
            jax.ShapeDtypeStruct((b, 1, n), jnp.int32),
        ],
    )(mc, cln_g_p, cln_b_p, cent_w_p, cent_b_p)

    cid_flat, tm_flat = _make_sc_select(b, n, ncl)(
        cs_p.reshape(b * _LANE), ids_p.reshape(b * n))

    state_t = state.reshape(b, 1, n)
    pooled_blk = pl.pallas_call(
        _k6_body,
        grid=(b, nb2),
        in_specs=[
            pl.BlockSpec((1, _BM2, ncls), lambda t, j: (t, j, 0)),
            pl.BlockSpec((1, n, ncls), lambda t, j: (t, 0, 0)),
            pl.BlockSpec((1, 1, n), lambda t, j: (t, 0, 0)),
        ],
        out_specs=pl.BlockSpec((1, 1, 1, _BM2), lambda t, j: (t, j, 0, 0)),
        out_shape=jax.ShapeDtypeStruct((b, nb2, 1, _BM2), f32),
    )(xs, xs, state_t)
    pooled = pooled_blk.reshape(b, n)

    d1 = mlp1_w.shape[1]
    y1 = pl.pallas_call(
        _k7a_body,
        grid=(d1 // _BC,),
        in_specs=[
            pl.BlockSpec((b, n), lambda c: (0, 0)),
            pl.BlockSpec((n,), lambda c: (0,)),
            pl.BlockSpec((n,), lambda c: (0,)),
            pl.BlockSpec((n, _BC), lambda c: (0, c)),
            pl.BlockSpec((_BC,), lambda c: (c,)),
            pl.BlockSpec((_BC,), lambda c: (c,)),
            pl.BlockSpec((_BC,), lambda c: (c,)),
        ],
        out_specs=pl.BlockSpec((b, _BC), lambda c: (0, c)),
        out_shape=jax.ShapeDtypeStruct((b, d1), f32),
    )(pooled, norm1_g, norm1_b, mlp1_w, mlp1_b, norm2_g, norm2_b)

    d2 = mlp2_w.shape[1]
    out = pl.pallas_call(
        _k7b_body,
        grid=(d2 // _BC,),
        in_specs=[
            pl.BlockSpec((b, d1), lambda c: (0, 0)),
            pl.BlockSpec((d1, _BC), lambda c: (0, c)),
            pl.BlockSpec((_BC,), lambda c: (c,)),
        ],
        out_specs=pl.BlockSpec((b, _BC), lambda c: (0, c)),
        out_shape=jax.ShapeDtypeStruct((b, d2), f32),
    )(y1, mlp2_w, mlp2_b)

    c_s = cs_p[:, 0, :ncl, None]
    t_mask = tm_flat.reshape(b, n, 1)
    c_id = cid_flat.reshape(b, _LANE)[:, :2]
    output = out.reshape(b, n, 2)
    return c_s, t_mask, c_id, output, adj
